# baseline (speedup 1.0000x reference)
import functools

import jax
import jax.numpy as jnp
from jax import lax
from jax.experimental import pallas as pl
from jax.experimental.pallas import tpu as pltpu
from jax.experimental.pallas import tpu_sc as plsc

N = 10000
E = 160000
D = 256
HALF = 128
G = 64

NSC = 2
NTILE = 16
CHUNK = 128
SB = 8
UPT = 10
EP = NTILE * UPT * SB * CHUNK
EPAD = EP - E
NCPT = UPT * SB
PADROWS = 16
ACC_ROWS = N + PADROWS
NZF = N // CHUNK
ZTAIL = N - NZF * CHUNK


def _leaky(v):
    return jnp.where(v > 0, v, 0.01 * v)


def _dot(a, b):
    return jnp.dot(a, b, preferred_element_type=jnp.float32,
                   precision=jax.lax.Precision.HIGHEST)


def _sc_agg_body(h2_hbm, src_hbm, dst_hbm, agg_hbm,
                 srcbuf, dstbuf, rows, acc_spmem, gsem, isem, ssem):
    c = lax.axis_index("c")
    s = lax.axis_index("s")

    for u in range(UPT):
        pltpu.async_copy(src_hbm.at[pl.ds((s + NTILE * u) * SB, SB)],
                         srcbuf.at[u], isem)
    pltpu.async_copy(dst_hbm.at[pl.ds(s * SB, SB)], dstbuf.at[0], isem)

    def _z(r, _):
        for l in range(HALF // 16):
            rows[0, r, pl.ds(l * 16, 16)] = jnp.zeros((16,), jnp.float32)
        return 0
    lax.fori_loop(0, CHUNK, _z, 0)
    nz = jnp.where(s < NZF - NTILE * (NZF // NTILE),
                   NZF // NTILE + 1, NZF // NTILE)

    def _zi(b, _):
        pltpu.async_copy(
            rows.at[0], acc_spmem.at[pl.ds((s + NTILE * b) * CHUNK, CHUNK)],
            gsem)
        return 0
    lax.fori_loop(0, nz, _zi, 0)

    @pl.when(s == NTILE - 1)
    def _ztail():
        pltpu.sync_copy(rows.at[0].at[pl.ds(0, ZTAIL)],
                        acc_spmem.at[pl.ds(NZF * CHUNK, ZTAIL)])

    def _zw(b, _):
        pltpu.make_async_copy(rows.at[0], acc_spmem.at[pl.ds(0, CHUNK)],
                              gsem).wait()
        return 0
    lax.fori_loop(0, nz, _zw, 0)

    for u in range(UPT):
        pltpu.make_async_copy(src_hbm.at[pl.ds(0, SB)], srcbuf.at[u],
                              isem).wait()

    def _ix(k, _):
        l = lax.rem(k, CHUNK // 16)
        r = lax.rem(k // (CHUNK // 16), SB)
        u = k // (SB * (CHUNK // 16))
        v = srcbuf[u, r, pl.ds(l * 16, 16)]
        srcbuf[u, r, pl.ds(l * 16, 16)] = v + v + c
        return 0
    lax.fori_loop(0, UPT * SB * (CHUNK // 16), _ix, 0)

    pltpu.async_copy(h2_hbm.at[srcbuf.at[0].at[0]], rows.at[0], gsem)
    plsc.subcore_barrier()

    def _outer(u, _):
        p = lax.rem(u, 2)
        np_ = lax.rem(u + 1, 2)

        @pl.when(u < UPT - 1)
        def _refill():
            pltpu.async_copy(dst_hbm.at[pl.ds((s + NTILE * (u + 1)) * SB,
                                              SB)],
                             dstbuf.at[np_], isem)
        pltpu.make_async_copy(dst_hbm.at[pl.ds(0, SB)], dstbuf.at[p],
                              isem).wait()
        for r in range(SB):
            if r == 0:
                @pl.when(u > 0)
                def _sw():
                    pltpu.make_async_copy(rows.at[1],
                                          acc_spmem.at[pl.ds(0, CHUNK)],
                                          ssem).wait()
            else:
                pltpu.make_async_copy(rows.at[(r + 1) & 1],
                                      acc_spmem.at[pl.ds(0, CHUNK)],
                                      ssem).wait()
            if r < SB - 1:
                pltpu.async_copy(h2_hbm.at[srcbuf.at[u].at[r + 1]],
                                 rows.at[(r + 1) & 1], gsem)
            else:
                @pl.when(u < UPT - 1)
                def _pref():
                    pltpu.async_copy(h2_hbm.at[srcbuf.at[u + 1].at[0]],
                                     rows.at[0], gsem)
            pltpu.make_async_copy(h2_hbm.at[pl.ds(0, CHUNK)],
                                  rows.at[r & 1], gsem).wait()
            pltpu.async_copy(rows.at[r & 1],
                             acc_spmem.at[dstbuf.at[p].at[r]], ssem,
                             add=True)
        return 0
    lax.fori_loop(0, UPT, _outer, 0)
    pltpu.make_async_copy(rows.at[1], acc_spmem.at[pl.ds(0, CHUNK)],
                          ssem).wait()
    plsc.subcore_barrier()

    def _dr(b, _):
        @pl.when(b >= 2)
        def _dw():
            pltpu.make_async_copy(rows.at[0], agg_hbm.at[c].at[pl.ds(0,
                                                                     CHUNK)],
                                  isem).wait()
        pltpu.sync_copy(
            acc_spmem.at[pl.ds((s + NTILE * b) * CHUNK, CHUNK)],
            rows.at[lax.rem(b, 2)])
        pltpu.async_copy(rows.at[lax.rem(b, 2)],
                         agg_hbm.at[c].at[pl.ds((s + NTILE * b) * CHUNK,
                                                CHUNK)], isem)
        return 0
    lax.fori_loop(0, nz, _dr, 0)

    def _dw2(b, _):
        pltpu.make_async_copy(rows.at[0], agg_hbm.at[c].at[pl.ds(0, CHUNK)],
                              isem).wait()
        return 0
    lax.fori_loop(0, jnp.minimum(nz, 2), _dw2, 0)

    @pl.when(s == NTILE - 1)
    def _dtail():
        pltpu.sync_copy(acc_spmem.at[pl.ds(NZF * CHUNK, ZTAIL)],
                        rows.at[0].at[pl.ds(0, ZTAIL)])
        pltpu.sync_copy(rows.at[0].at[pl.ds(0, ZTAIL)],
                        agg_hbm.at[c].at[pl.ds(NZF * CHUNK, ZTAIL)])


@functools.cache
def _sc_agg_kernel():
    return pl.kernel(
        _sc_agg_body,
        out_type=jax.ShapeDtypeStruct((NSC, N, HALF), jnp.float32),
        mesh=plsc.VectorSubcoreMesh(core_axis_name="c", subcore_axis_name="s",
                                    num_cores=NSC, num_subcores=NTILE),
        scratch_types=[
            pltpu.VMEM((UPT, SB, CHUNK), jnp.int32),
            pltpu.VMEM((2, SB, CHUNK), jnp.int32),
            pltpu.VMEM((2, CHUNK, HALF), jnp.float32),
            pltpu.VMEM_SHARED((ACC_ROWS, HALF), jnp.float32),
            pltpu.SemaphoreType.DMA,
            pltpu.SemaphoreType.DMA,
            pltpu.SemaphoreType.DMA,
        ],
    )


def _sc_agg(h2, src2d, dst2d):
    return _sc_agg_kernel()(h2, src2d, dst2d)


R = 2000
NG = N // R


def _mlp_block_z(h_ref, agg_ref, w1_ref, b1_ref, w2_ref, b2_ref):
    z = h_ref[...] + jnp.concatenate([agg_ref[0], agg_ref[1]], axis=1)
    t = _leaky(_dot(z, w1_ref[...])
               + b1_ref[...][None, :])
    y = _dot(t, w2_ref[...]) + b2_ref[...][None, :]
    return _leaky(y)


def _tc_mlp_body(h_ref, agg_ref, w1_ref, b1_ref, w2_ref, b2_ref, out_ref):
    out_ref[...] = _mlp_block_z(h_ref, agg_ref, w1_ref, b1_ref, w2_ref, b2_ref)


def _tc_mlp(h, agg, w1, b1, w2, b2):
    return pl.pallas_call(
        _tc_mlp_body,
        grid=(NG,),
        in_specs=[
            pl.BlockSpec((R, D), lambda i: (i, 0)),
            pl.BlockSpec((NSC, R, HALF), lambda i: (0, i, 0)),
            pl.BlockSpec((D, D), lambda i: (0, 0)),
            pl.BlockSpec((D,), lambda i: (0,)),
            pl.BlockSpec((D, D), lambda i: (0, 0)),
            pl.BlockSpec((D,), lambda i: (0,)),
        ],
        out_specs=pl.BlockSpec((R, D), lambda i: (i, 0)),
        out_shape=jax.ShapeDtypeStruct((N, D), jnp.float32),
        compiler_params=pltpu.CompilerParams(
            dimension_semantics=("arbitrary",)),
    )(h, agg, w1, b1, w2, b2)


def _tc_mlp_pool_body(h_ref, agg_ref, w1_ref, b1_ref, w2_ref, b2_ref,
                      batch_ref, hw1_ref, hb1_ref, hw2_ref, hb2_ref,
                      out_ref, pooled_acc, cnt_acc):
    i = pl.program_id(0)

    @pl.when(i == 0)
    def _init():
        pooled_acc[...] = jnp.zeros((G, D), jnp.float32)
        cnt_acc[...] = jnp.zeros((G,), jnp.float32)

    y = _mlp_block_z(h_ref, agg_ref, w1_ref, b1_ref, w2_ref, b2_ref)
    batch_blk = batch_ref[0, 0, :]
    onehot = (batch_blk[None, :] ==
              lax.broadcasted_iota(jnp.int32, (G, R), 0)).astype(jnp.float32)
    pooled_acc[...] += _dot(onehot, y)
    cnt_acc[...] += jnp.sum(onehot, axis=1)

    @pl.when(i == NG - 1)
    def _fin():
        pooled = pooled_acc[...] / jnp.maximum(cnt_acc[...], 1.0)[:, None]
        zh = _leaky(_dot(pooled, hw1_ref[...])
                    + hb1_ref[...][None, :])
        out_ref[...] = _dot(zh, hw2_ref[...]) + hb2_ref[...][None, :]


def _tc_mlp_pool(h, agg, w1, b1, w2, b2, batch, hw1, hb1, hw2, hb2):
    return pl.pallas_call(
        _tc_mlp_pool_body,
        grid=(NG,),
        in_specs=[
            pl.BlockSpec((R, D), lambda i: (i, 0)),
            pl.BlockSpec((NSC, R, HALF), lambda i: (0, i, 0)),
            pl.BlockSpec((D, D), lambda i: (0, 0)),
            pl.BlockSpec((D,), lambda i: (0,)),
            pl.BlockSpec((D, D), lambda i: (0, 0)),
            pl.BlockSpec((D,), lambda i: (0,)),
            pl.BlockSpec((1, 1, R), lambda i: (i, 0, 0)),
            pl.BlockSpec((D, D), lambda i: (0, 0)),
            pl.BlockSpec((D,), lambda i: (0,)),
            pl.BlockSpec((D, 1), lambda i: (0, 0)),
            pl.BlockSpec((1,), lambda i: (0,)),
        ],
        out_specs=pl.BlockSpec((G, 1), lambda i: (0, 0)),
        out_shape=jax.ShapeDtypeStruct((G, 1), jnp.float32),
        scratch_shapes=[
            pltpu.VMEM((G, D), jnp.float32),
            pltpu.VMEM((G,), jnp.float32),
        ],
        compiler_params=pltpu.CompilerParams(
            dimension_semantics=("arbitrary",)),
    )(h, agg, w1, b1, w2, b2, batch.reshape(NG, 1, R), hw1, hb1, hw2, hb2)


def kernel(x, edge_index, batch,
           l0_W1, l0_b1, l0_W2, l0_b2,
           l1_W1, l1_b1, l1_W2, l1_b2,
           l2_W1, l2_b1, l2_W2, l2_b2,
           head_W1, head_b1, head_W2, head_b2):
    ip = jnp.arange(EPAD, dtype=jnp.int32)
    src2d = jnp.concatenate([edge_index[0], ip % N]).reshape(EP // CHUNK,
                                                            CHUNK)
    dst2d = jnp.concatenate([edge_index[1], N + (ip % PADROWS)]).reshape(
        EP // CHUNK, CHUNK)

    h = x
    agg = _sc_agg(h.reshape(2 * N, HALF), src2d, dst2d)
    h = _tc_mlp(h, agg, l0_W1, l0_b1, l0_W2, l0_b2)
    agg = _sc_agg(h.reshape(2 * N, HALF), src2d, dst2d)
    h = _tc_mlp(h, agg, l1_W1, l1_b1, l1_W2, l1_b2)
    agg = _sc_agg(h.reshape(2 * N, HALF), src2d, dst2d)
    out = _tc_mlp_pool(h, agg, l2_W1, l2_b1, l2_W2, l2_b2,
                       batch, head_W1, head_b1, head_W2, head_b2)
    return out


# speedup vs baseline: 1.2468x; 1.2468x over previous
import functools

import jax
import jax.numpy as jnp
from jax import lax
from jax.experimental import pallas as pl
from jax.experimental.pallas import tpu as pltpu
from jax.experimental.pallas import tpu_sc as plsc

N = 10000
E = 160000
D = 256
HALF = 128
G = 64

NSC = 2
NTILE = 16
CHUNK = 128
SB = 8
UPT = 10
EP = NTILE * UPT * SB * CHUNK
EPAD = EP - E
NCPT = UPT * SB
PADROWS = 16
ACC_ROWS = N + PADROWS
NZF = N // CHUNK
ZTAIL = N - NZF * CHUNK


def _leaky(v):
    return jnp.where(v > 0, v, 0.01 * v)


def _dot(a, b):
    return jnp.dot(a, b, preferred_element_type=jnp.float32,
                   precision=None)


def _sc_agg_body(h2_hbm, src_hbm, dst_hbm, agg_hbm,
                 srcbuf, dstbuf, rows, acc_spmem, gsem, isem, ssem,
                 interleaved):
    c = lax.axis_index("c")
    s = lax.axis_index("s")

    for u in range(UPT):
        pltpu.async_copy(src_hbm.at[pl.ds((s + NTILE * u) * SB, SB)],
                         srcbuf.at[u], isem)
    pltpu.async_copy(dst_hbm.at[pl.ds(s * SB, SB)], dstbuf.at[0], isem)

    def _z(r, _):
        for l in range(HALF // 16):
            rows[0, r, pl.ds(l * 16, 16)] = jnp.zeros((16,), jnp.float32)
        return 0
    lax.fori_loop(0, CHUNK, _z, 0)
    nz = jnp.where(s < NZF - NTILE * (NZF // NTILE),
                   NZF // NTILE + 1, NZF // NTILE)

    def _zi(b, _):
        pltpu.async_copy(
            rows.at[0], acc_spmem.at[pl.ds((s + NTILE * b) * CHUNK, CHUNK)],
            gsem)
        return 0
    lax.fori_loop(0, nz, _zi, 0)

    @pl.when(s == NTILE - 1)
    def _ztail():
        pltpu.sync_copy(rows.at[0].at[pl.ds(0, ZTAIL)],
                        acc_spmem.at[pl.ds(NZF * CHUNK, ZTAIL)])

    def _zw(b, _):
        pltpu.make_async_copy(rows.at[0], acc_spmem.at[pl.ds(0, CHUNK)],
                              gsem).wait()
        return 0
    lax.fori_loop(0, nz, _zw, 0)

    for u in range(UPT):
        pltpu.make_async_copy(src_hbm.at[pl.ds(0, SB)], srcbuf.at[u],
                              isem).wait()

    base = c * N

    def _ix(k, _):
        l = lax.rem(k, CHUNK // 16)
        r = lax.rem(k // (CHUNK // 16), SB)
        u = k // (SB * (CHUNK // 16))
        v = srcbuf[u, r, pl.ds(l * 16, 16)]
        if interleaved:
            srcbuf[u, r, pl.ds(l * 16, 16)] = v + v + c
        else:
            srcbuf[u, r, pl.ds(l * 16, 16)] = v + base
        return 0
    lax.fori_loop(0, UPT * SB * (CHUNK // 16), _ix, 0)

    pltpu.async_copy(h2_hbm.at[srcbuf.at[0].at[0]], rows.at[0], gsem)
    plsc.subcore_barrier()

    def _outer(u, _):
        p = lax.rem(u, 2)
        np_ = lax.rem(u + 1, 2)

        @pl.when(u < UPT - 1)
        def _refill():
            pltpu.async_copy(dst_hbm.at[pl.ds((s + NTILE * (u + 1)) * SB,
                                              SB)],
                             dstbuf.at[np_], isem)
        pltpu.make_async_copy(dst_hbm.at[pl.ds(0, SB)], dstbuf.at[p],
                              isem).wait()
        for r in range(SB):
            if r == 0:
                @pl.when(u > 0)
                def _sw():
                    pltpu.make_async_copy(rows.at[1],
                                          acc_spmem.at[pl.ds(0, CHUNK)],
                                          ssem).wait()
            else:
                pltpu.make_async_copy(rows.at[(r + 1) & 1],
                                      acc_spmem.at[pl.ds(0, CHUNK)],
                                      ssem).wait()
            if r < SB - 1:
                pltpu.async_copy(h2_hbm.at[srcbuf.at[u].at[r + 1]],
                                 rows.at[(r + 1) & 1], gsem)
            else:
                @pl.when(u < UPT - 1)
                def _pref():
                    pltpu.async_copy(h2_hbm.at[srcbuf.at[u + 1].at[0]],
                                     rows.at[0], gsem)
            pltpu.make_async_copy(h2_hbm.at[pl.ds(0, CHUNK)],
                                  rows.at[r & 1], gsem).wait()
            pltpu.async_copy(rows.at[r & 1],
                             acc_spmem.at[dstbuf.at[p].at[r]], ssem,
                             add=True)
        return 0
    lax.fori_loop(0, UPT, _outer, 0)
    pltpu.make_async_copy(rows.at[1], acc_spmem.at[pl.ds(0, CHUNK)],
                          ssem).wait()
    plsc.subcore_barrier()

    def _dr(b, _):
        @pl.when(b >= 2)
        def _dw():
            pltpu.make_async_copy(rows.at[0], agg_hbm.at[c].at[pl.ds(0,
                                                                     CHUNK)],
                                  isem).wait()
        pltpu.sync_copy(
            acc_spmem.at[pl.ds((s + NTILE * b) * CHUNK, CHUNK)],
            rows.at[lax.rem(b, 2)])
        pltpu.async_copy(rows.at[lax.rem(b, 2)],
                         agg_hbm.at[c].at[pl.ds((s + NTILE * b) * CHUNK,
                                                CHUNK)], isem)
        return 0
    lax.fori_loop(0, nz, _dr, 0)

    def _dw2(b, _):
        pltpu.make_async_copy(rows.at[0], agg_hbm.at[c].at[pl.ds(0, CHUNK)],
                              isem).wait()
        return 0
    lax.fori_loop(0, jnp.minimum(nz, 2), _dw2, 0)

    @pl.when(s == NTILE - 1)
    def _dtail():
        pltpu.sync_copy(acc_spmem.at[pl.ds(NZF * CHUNK, ZTAIL)],
                        rows.at[0].at[pl.ds(0, ZTAIL)])
        pltpu.sync_copy(rows.at[0].at[pl.ds(0, ZTAIL)],
                        agg_hbm.at[c].at[pl.ds(NZF * CHUNK, ZTAIL)])


@functools.cache
def _sc_agg_kernel(interleaved):
    return pl.kernel(
        functools.partial(_sc_agg_body, interleaved=interleaved),
        out_type=jax.ShapeDtypeStruct((NSC, N, HALF), jnp.float32),
        mesh=plsc.VectorSubcoreMesh(core_axis_name="c", subcore_axis_name="s",
                                    num_cores=NSC, num_subcores=NTILE),
        scratch_types=[
            pltpu.VMEM((UPT, SB, CHUNK), jnp.int32),
            pltpu.VMEM((2, SB, CHUNK), jnp.int32),
            pltpu.VMEM((2, CHUNK, HALF), jnp.float32),
            pltpu.VMEM_SHARED((ACC_ROWS, HALF), jnp.float32),
            pltpu.SemaphoreType.DMA,
            pltpu.SemaphoreType.DMA,
            pltpu.SemaphoreType.DMA,
        ],
    )


def _sc_agg(h2, src2d, dst2d, interleaved):
    return _sc_agg_kernel(interleaved)(h2, src2d, dst2d)


R = 2000
NG = N // R


def _mlp_block_z(h_ref, agg_ref, w1_ref, b1_ref, w2_ref, b2_ref, split_in):
    if split_in:
        z = jnp.concatenate([h_ref[0] + agg_ref[0], h_ref[1] + agg_ref[1]],
                            axis=1)
    else:
        z = h_ref[...] + jnp.concatenate([agg_ref[0], agg_ref[1]], axis=1)
    t = _leaky(_dot(z, w1_ref[...])
               + b1_ref[...][None, :])
    y = _dot(t, w2_ref[...]) + b2_ref[...][None, :]
    return _leaky(y)


def _tc_mlp_body(h_ref, agg_ref, w1_ref, b1_ref, w2_ref, b2_ref, out_ref,
                 split_in):
    y = _mlp_block_z(h_ref, agg_ref, w1_ref, b1_ref, w2_ref, b2_ref,
                     split_in)
    out_ref[0] = y[:, :HALF]
    out_ref[1] = y[:, HALF:]


def _tc_mlp(h, agg, w1, b1, w2, b2, split_in):
    h_spec = (pl.BlockSpec((NSC, R, HALF), lambda i: (0, i, 0)) if split_in
              else pl.BlockSpec((R, D), lambda i: (i, 0)))
    return pl.pallas_call(
        functools.partial(_tc_mlp_body, split_in=split_in),
        grid=(NG,),
        in_specs=[
            h_spec,
            pl.BlockSpec((NSC, R, HALF), lambda i: (0, i, 0)),
            pl.BlockSpec((D, D), lambda i: (0, 0)),
            pl.BlockSpec((D,), lambda i: (0,)),
            pl.BlockSpec((D, D), lambda i: (0, 0)),
            pl.BlockSpec((D,), lambda i: (0,)),
        ],
        out_specs=pl.BlockSpec((NSC, R, HALF), lambda i: (0, i, 0)),
        out_shape=jax.ShapeDtypeStruct((NSC, N, HALF), jnp.float32),
        compiler_params=pltpu.CompilerParams(
            dimension_semantics=("arbitrary",)),
    )(h, agg, w1, b1, w2, b2)


def _tc_mlp_pool_body(h_ref, agg_ref, w1_ref, b1_ref, w2_ref, b2_ref,
                      batch_ref, hw1_ref, hb1_ref, hw2_ref, hb2_ref,
                      out_ref, pooled_acc, cnt_acc):
    i = pl.program_id(0)

    @pl.when(i == 0)
    def _init():
        pooled_acc[...] = jnp.zeros((G, D), jnp.float32)
        cnt_acc[...] = jnp.zeros((G,), jnp.float32)

    y = _mlp_block_z(h_ref, agg_ref, w1_ref, b1_ref, w2_ref, b2_ref,
                     split_in=True)
    batch_blk = batch_ref[0, 0, :]
    onehot = (batch_blk[None, :] ==
              lax.broadcasted_iota(jnp.int32, (G, R), 0)).astype(jnp.float32)
    pooled_acc[...] += _dot(onehot, y)
    cnt_acc[...] += jnp.sum(onehot, axis=1)

    @pl.when(i == NG - 1)
    def _fin():
        pooled = pooled_acc[...] / jnp.maximum(cnt_acc[...], 1.0)[:, None]
        zh = _leaky(_dot(pooled, hw1_ref[...])
                    + hb1_ref[...][None, :])
        out_ref[...] = _dot(zh, hw2_ref[...]) + hb2_ref[...][None, :]


def _tc_mlp_pool(h, agg, w1, b1, w2, b2, batch, hw1, hb1, hw2, hb2):
    return pl.pallas_call(
        _tc_mlp_pool_body,
        grid=(NG,),
        in_specs=[
            pl.BlockSpec((NSC, R, HALF), lambda i: (0, i, 0)),
            pl.BlockSpec((NSC, R, HALF), lambda i: (0, i, 0)),
            pl.BlockSpec((D, D), lambda i: (0, 0)),
            pl.BlockSpec((D,), lambda i: (0,)),
            pl.BlockSpec((D, D), lambda i: (0, 0)),
            pl.BlockSpec((D,), lambda i: (0,)),
            pl.BlockSpec((1, 1, R), lambda i: (i, 0, 0)),
            pl.BlockSpec((D, D), lambda i: (0, 0)),
            pl.BlockSpec((D,), lambda i: (0,)),
            pl.BlockSpec((D, 1), lambda i: (0, 0)),
            pl.BlockSpec((1,), lambda i: (0,)),
        ],
        out_specs=pl.BlockSpec((G, 1), lambda i: (0, 0)),
        out_shape=jax.ShapeDtypeStruct((G, 1), jnp.float32),
        scratch_shapes=[
            pltpu.VMEM((G, D), jnp.float32),
            pltpu.VMEM((G,), jnp.float32),
        ],
        compiler_params=pltpu.CompilerParams(
            dimension_semantics=("arbitrary",)),
    )(h, agg, w1, b1, w2, b2, batch.reshape(NG, 1, R), hw1, hb1, hw2, hb2)


def kernel(x, edge_index, batch,
           l0_W1, l0_b1, l0_W2, l0_b2,
           l1_W1, l1_b1, l1_W2, l1_b2,
           l2_W1, l2_b1, l2_W2, l2_b2,
           head_W1, head_b1, head_W2, head_b2):
    ip = jnp.arange(EPAD, dtype=jnp.int32)
    src2d = jnp.concatenate([edge_index[0], ip % N]).reshape(EP // CHUNK,
                                                            CHUNK)
    dst2d = jnp.concatenate([edge_index[1], N + (ip % PADROWS)]).reshape(
        EP // CHUNK, CHUNK)

    agg = _sc_agg(x.reshape(2 * N, HALF), src2d, dst2d, interleaved=True)
    h = _tc_mlp(x, agg, l0_W1, l0_b1, l0_W2, l0_b2, split_in=False)
    agg = _sc_agg(h.reshape(2 * N, HALF), src2d, dst2d, interleaved=False)
    h = _tc_mlp(h, agg, l1_W1, l1_b1, l1_W2, l1_b2, split_in=True)
    agg = _sc_agg(h.reshape(2 * N, HALF), src2d, dst2d, interleaved=False)
    out = _tc_mlp_pool(h, agg, l2_W1, l2_b1, l2_W2, l2_b2,
                       batch, head_W1, head_b1, head_W2, head_b2)
    return out
